# baseline (speedup 1.0000x reference)
import jax
import jax.numpy as jnp
from jax import lax
from jax.experimental import pallas as pl
from jax.experimental.pallas import tpu as pltpu
from jax.experimental.pallas import tpu_sc as plsc

_COLS = (0, 1, 5)
_LEAD = 8


def _make_sc_kernel(n_rows: int, dtype):
    info = plsc.get_sparse_core_info()
    nc, ns, lanes = info.num_cores, info.num_subcores, info.num_lanes
    nw = nc * ns
    chunk = n_rows // nw
    assert chunk % lanes == 0 and chunk % 8 == 0

    n_streams = 4
    piece = chunk // n_streams

    def body(x_hbm, out_hbm, rows_v, out_v, sem_in, sem_out):
        wid = lax.axis_index("s") * nc + lax.axis_index("c")
        base = wid * chunk
        stage = [
            pltpu.make_async_copy(
                x_hbm.at[pl.ds(base + h * piece, piece), pl.ds(0, _LEAD)],
                rows_v.at[pl.ds(h * piece, piece)],
                sem_in,
            )
            for h in range(n_streams)
        ]
        stage[0].start()
        stage[1].start()
        lane_iota = lax.iota(jnp.int32, lanes)

        cidxs = [jnp.full((lanes,), c, jnp.int32) for c in _COLS]

        def extract(t_begin, t_end):
            group = 8
            for t0 in range(t_begin, t_end, group):
                batch = []
                for i in range(len(_COLS)):
                    for t in range(t0, min(t0 + group, t_end)):
                        ridx = lane_iota + t * lanes
                        batch.append(
                            (i, t, plsc.load_gather(rows_v, [ridx, cidxs[i]]))
                        )
                for i, t, vals in batch:
                    out_v[i, pl.ds(t * lanes, lanes)] = vals

        drain = [
            pltpu.make_async_copy(
                out_v.at[:, pl.ds(h * piece, piece)],
                out_hbm.at[:, pl.ds(base + h * piece, piece)],
                sem_out,
            )
            for h in range(n_streams)
        ]
        for h in range(n_streams):
            with jax.named_scope(f"stage_wait{h}"):
                stage[h].wait()
            if h + 2 < n_streams:
                stage[h + 2].start()
            with jax.named_scope(f"extract{h}"):
                extract(h * piece // lanes, (h + 1) * piece // lanes)
            drain[h].start()
        with jax.named_scope("drain_wait"):
            for h in range(n_streams):
                drain[h].wait()

    return pl.kernel(
        body,
        out_type=jax.ShapeDtypeStruct((len(_COLS), n_rows), dtype),
        mesh=plsc.VectorSubcoreMesh(core_axis_name="c", subcore_axis_name="s"),
        scratch_types=[
            pltpu.VMEM((chunk, _LEAD), jnp.float32),
            pltpu.VMEM((len(_COLS), chunk), jnp.float32),
            pltpu.SemaphoreType.DMA,
            pltpu.SemaphoreType.DMA,
        ],
        compiler_params=pltpu.CompilerParams(
            use_tc_tiling_on_sc=False, needs_layout_passes=False
        ),
    )


def kernel(x):
    n_rows = x.shape[0]
    return _make_sc_kernel(n_rows, x.dtype)(x)


# speedup vs baseline: 1.0390x; 1.0390x over previous
import jax
import jax.numpy as jnp
from jax import lax
from jax.experimental import pallas as pl
from jax.experimental.pallas import tpu as pltpu
from jax.experimental.pallas import tpu_sc as plsc

_COLS = (0, 1, 5)
_LEAD = 8


def _make_sc_kernel(n_rows: int, dtype):
    info = plsc.get_sparse_core_info()
    nc, ns, lanes = info.num_cores, info.num_subcores, info.num_lanes
    nw = nc * ns
    chunk = n_rows // nw
    assert chunk % lanes == 0 and chunk % 8 == 0

    sizes = [32, 32, 64, 64, 64, 64, 96, 96]
    assert sum(sizes) == chunk
    starts = [sum(sizes[:h]) for h in range(len(sizes))]
    n_streams = len(sizes)

    def body(x_hbm, out_hbm, rows_v, out_v, *sems):
        sems_in, sem_out = sems[:n_streams], sems[n_streams]
        wid = lax.axis_index("s") * nc + lax.axis_index("c")
        base = wid * chunk
        stage = [
            pltpu.make_async_copy(
                x_hbm.at[pl.ds(base + starts[h], sizes[h]), pl.ds(0, _LEAD)],
                rows_v.at[pl.ds(starts[h], sizes[h])],
                sems_in[h],
            )
            for h in range(n_streams)
        ]
        for cp in stage:
            cp.start()
        lane_iota = lax.iota(jnp.int32, lanes)

        cidxs = [jnp.full((lanes,), c, jnp.int32) for c in _COLS]

        def extract(t_begin, t_end):
            group = 8
            for t0 in range(t_begin, t_end, group):
                batch = []
                for i in range(len(_COLS)):
                    for t in range(t0, min(t0 + group, t_end)):
                        ridx = lane_iota + t * lanes
                        batch.append(
                            (i, t, plsc.load_gather(rows_v, [ridx, cidxs[i]]))
                        )
                for i, t, vals in batch:
                    out_v[i, pl.ds(t * lanes, lanes)] = vals

        drain = [
            pltpu.make_async_copy(
                out_v.at[:, pl.ds(starts[h], sizes[h])],
                out_hbm.at[:, pl.ds(base + starts[h], sizes[h])],
                sem_out,
            )
            for h in range(n_streams)
        ]
        for h in range(n_streams):
            with jax.named_scope(f"stage_wait{h}"):
                stage[h].wait()
            with jax.named_scope(f"extract{h}"):
                extract(starts[h] // lanes, (starts[h] + sizes[h]) // lanes)
            drain[h].start()
        with jax.named_scope("drain_wait"):
            for h in range(n_streams):
                drain[h].wait()

    return pl.kernel(
        body,
        out_type=jax.ShapeDtypeStruct((len(_COLS), n_rows), dtype),
        mesh=plsc.VectorSubcoreMesh(core_axis_name="c", subcore_axis_name="s"),
        scratch_types=[
            pltpu.VMEM((chunk, _LEAD), jnp.float32),
            pltpu.VMEM((len(_COLS), chunk), jnp.float32),
        ]
        + [pltpu.SemaphoreType.DMA] * (n_streams + 1),
        compiler_params=pltpu.CompilerParams(
            use_tc_tiling_on_sc=False, needs_layout_passes=False
        ),
    )


def kernel(x):
    n_rows = x.shape[0]
    return _make_sc_kernel(n_rows, x.dtype)(x)


# speedup vs baseline: 1.0591x; 1.0194x over previous
import jax
import jax.numpy as jnp
from jax import lax
from jax.experimental import pallas as pl
from jax.experimental.pallas import tpu as pltpu
from jax.experimental.pallas import tpu_sc as plsc

_COLS = (0, 1, 5)
_LEAD = 8


def _make_sc_kernel(n_rows: int, dtype):
    info = plsc.get_sparse_core_info()
    nc, ns, lanes = info.num_cores, info.num_subcores, info.num_lanes
    nw = nc * ns
    chunk = n_rows // nw
    assert chunk % lanes == 0 and chunk % 8 == 0

    sizes = [64, 96, 160, 192]
    assert sum(sizes) == chunk
    starts = [sum(sizes[:h]) for h in range(len(sizes))]
    n_streams = len(sizes)

    def body(x_hbm, out_hbm, rows_v, out_v, *sems):
        sems_in, sem_out = sems[:n_streams], sems[n_streams]
        wid = lax.axis_index("s") * nc + lax.axis_index("c")
        base = wid * chunk
        stage = [
            pltpu.make_async_copy(
                x_hbm.at[pl.ds(base + starts[h], sizes[h]), pl.ds(0, _LEAD)],
                rows_v.at[pl.ds(starts[h], sizes[h])],
                sems_in[h],
            )
            for h in range(n_streams)
        ]
        for cp in stage:
            cp.start()
        lane_iota = lax.iota(jnp.int32, lanes)

        cidxs = [jnp.full((lanes,), c, jnp.int32) for c in _COLS]

        def extract(t_begin, t_end):
            group = 8
            for t0 in range(t_begin, t_end, group):
                batch = []
                for i in range(len(_COLS)):
                    for t in range(t0, min(t0 + group, t_end)):
                        ridx = lane_iota + t * lanes
                        batch.append(
                            (i, t, plsc.load_gather(rows_v, [ridx, cidxs[i]]))
                        )
                for i, t, vals in batch:
                    out_v[i, pl.ds(t * lanes, lanes)] = vals

        drain = [
            pltpu.make_async_copy(
                out_v.at[:, pl.ds(starts[h], sizes[h])],
                out_hbm.at[:, pl.ds(base + starts[h], sizes[h])],
                sem_out,
            )
            for h in range(n_streams)
        ]
        for h in range(n_streams):
            stage[h].wait()
            extract(starts[h] // lanes, (starts[h] + sizes[h]) // lanes)
            drain[h].start()
        for h in range(n_streams):
            drain[h].wait()

    return pl.kernel(
        body,
        out_type=jax.ShapeDtypeStruct((len(_COLS), n_rows), dtype),
        mesh=plsc.VectorSubcoreMesh(core_axis_name="c", subcore_axis_name="s"),
        scratch_types=[
            pltpu.VMEM((chunk, _LEAD), jnp.float32),
            pltpu.VMEM((len(_COLS), chunk), jnp.float32),
        ]
        + [pltpu.SemaphoreType.DMA] * (n_streams + 1),
        compiler_params=pltpu.CompilerParams(
            use_tc_tiling_on_sc=False, needs_layout_passes=False
        ),
    )


def kernel(x):
    n_rows = x.shape[0]
    return _make_sc_kernel(n_rows, x.dtype)(x)
